# Initial kernel scaffold; baseline (speedup 1.0000x reference)
import jax
import jax.numpy as jnp
from jax.experimental import pallas as pl


def kernel(x, edge_attr, edge_index, batch, n1_Wn, n1_an, n1_We, n1_ae, e1_Wn, e1_We, e1_a, n2_Wn, n2_an, n2_We, n2_ae, Wr, br):
    raise NotImplementedError("write your pallas kernel here")


# speedup vs baseline: 3.6501x; 3.6501x over previous
import jax
import jax.numpy as jnp
from jax.experimental import pallas as pl

N = 10000
E = 320000
H = 64
NUM_GRAPHS = 16
BE = 3200
EPS = 1e-16


def _leaky(x, slope=0.2):
    return jnp.where(x >= 0, x, slope * x)


def _elu(x):
    return jnp.where(x > 0, x, jnp.expm1(x))


def _elu_k(x):
    return jnp.where(x > 0, x, jnp.exp(x) - 1.0)


def _edge_pass1_body(eattr_ref, h1d_ref, h1s_ref, w_ref, out_ref):
    eattr = eattr_ref[...]
    h1d = h1d_ref[...]
    h1s = h1s_ref[...]
    an_d = w_ref[0:1, 0:H]
    an_s = w_ref[1:2, 0:H]
    ae_d = w_ref[2:3, 0:H]
    w16 = w_ref[3:4, 0:16]
    a1dd = jnp.sum(h1d * an_d, axis=1, keepdims=True)
    a1ss = jnp.sum(h1s * an_s, axis=1, keepdims=True)
    a1ed = jnp.sum(h1d * ae_d, axis=1, keepdims=True)
    b1 = jnp.sum(eattr * w16, axis=1, keepdims=True)
    wn = jnp.exp(_leaky(a1dd + a1ss))
    we = jnp.exp(_leaky(a1ed + b1))
    zero = jnp.zeros((eattr.shape[0], 128 - H - 16 - 2), jnp.float32)
    out_ref[...] = jnp.concatenate(
        [wn * h1s, we * eattr, wn, we, zero], axis=1)


def _edge_pass2_body(eattr_ref, hns_ref, hnd_ref, h2s_ref, h2d_ref, w_ref,
                     out_ref):
    eattr = eattr_ref[...]
    hns = hns_ref[...]
    hnd = hnd_ref[...]
    h2s = h2s_ref[...]
    h2d = h2d_ref[...]
    e1We = w_ref[0:16, 0:H]
    ca = w_ref[16:17, 0:H]
    w1 = w_ref[17:18, 0:H]
    wgb = w_ref[18:19, 0:H]
    a2sv = w_ref[19:20, 0:H]
    wce = w_ref[20:21, 0:16]
    an2d = w_ref[21:22, 0:H]
    ae2d = w_ref[22:23, 0:H]
    c_e = jnp.sum(eattr * wce, axis=1, keepdims=True)
    cns = jnp.sum(hns * ca, axis=1, keepdims=True)
    cnd = jnp.sum(hnd * ca, axis=1, keepdims=True)
    ls = _leaky(c_e + cns)
    ld = _leaky(c_e + cnd)
    m2 = jnp.maximum(ls, ld)
    es = jnp.exp(ls - m2)
    ed = jnp.exp(ld - m2)
    den2 = es + ed + EPS
    enp = _elu_k((es * hns + ed * hnd) / den2)
    g = _elu_k(jnp.dot(eattr, e1We, preferred_element_type=jnp.float32))
    b2 = jnp.sum(enp * w1 + g * wgb, axis=1, keepdims=True)
    a2dd = jnp.sum(h2d * an2d, axis=1, keepdims=True)
    a2ed = jnp.sum(h2d * ae2d, axis=1, keepdims=True)
    a2ss = jnp.sum(h2s * a2sv, axis=1, keepdims=True)
    wn2 = jnp.exp(_leaky(a2dd + a2ss))
    we2 = jnp.exp(_leaky(a2ed + b2))
    zero = jnp.zeros((eattr.shape[0], 256 - 3 * H - 2), jnp.float32)
    out_ref[...] = jnp.concatenate(
        [wn2 * h2s, we2 * enp, we2 * g, wn2, we2, zero], axis=1)


def _seg_sum(v, seg, n):
    return jax.ops.segment_sum(v, seg, num_segments=n)


def kernel(x, edge_attr, edge_index, batch, n1_Wn, n1_an, n1_We, n1_ae, e1_Wn, e1_We, e1_a, n2_Wn, n2_an, n2_We, n2_ae, Wr, br):
    src, dst = edge_index[0], edge_index[1]
    nb = E // BE
    h1 = x @ n1_Wn
    w1pack = jnp.zeros((8, 128), jnp.float32)
    w1pack = w1pack.at[0, 0:H].set(n1_an[:H])
    w1pack = w1pack.at[1, 0:H].set(n1_an[H:])
    w1pack = w1pack.at[2, 0:H].set(n1_ae[:H])
    w1pack = w1pack.at[3, 0:16].set(n1_We @ n1_ae[H:])

    espec = pl.BlockSpec((BE, 16), lambda i: (i, 0))
    hspec = pl.BlockSpec((BE, H), lambda i: (i, 0))
    out1 = pl.pallas_call(
        _edge_pass1_body,
        grid=(nb,),
        in_specs=[espec, hspec, hspec,
                  pl.BlockSpec((8, 128), lambda i: (0, 0))],
        out_specs=pl.BlockSpec((BE, 128), lambda i: (i, 0)),
        out_shape=jax.ShapeDtypeStruct((E, 128), jnp.float32),
    )(edge_attr, h1[dst], h1[src], w1pack)

    agg1 = _seg_sum(out1, dst, N)
    den_n = agg1[:, H + 16]
    den_e = agg1[:, H + 17]
    nagg = agg1[:, :H] / (den_n + EPS)[:, None]
    eagg = (agg1[:, H:H + 16] @ n1_We) / (den_e + EPS)[:, None]
    x1 = _elu(jnp.concatenate([nagg, eagg], axis=1))

    hn = x1 @ e1_Wn
    h2 = x1 @ n2_Wn
    ae2v = n2_ae[H:]
    W_np, W_g = n2_We[:H], n2_We[H:]
    w2pack = jnp.zeros((24, 128), jnp.float32)
    w2pack = w2pack.at[0:16, 0:H].set(e1_We)
    w2pack = w2pack.at[16, 0:H].set(e1_a[H:])
    w2pack = w2pack.at[17, 0:H].set(W_np @ ae2v)
    w2pack = w2pack.at[18, 0:H].set(W_g @ ae2v)
    w2pack = w2pack.at[19, 0:H].set(n2_an[H:])
    w2pack = w2pack.at[20, 0:16].set(e1_We @ e1_a[:H])
    w2pack = w2pack.at[21, 0:H].set(n2_an[:H])
    w2pack = w2pack.at[22, 0:H].set(n2_ae[:H])

    out2 = pl.pallas_call(
        _edge_pass2_body,
        grid=(nb,),
        in_specs=[espec, hspec, hspec, hspec, hspec,
                  pl.BlockSpec((24, 128), lambda i: (0, 0))],
        out_specs=pl.BlockSpec((BE, 256), lambda i: (i, 0)),
        out_shape=jax.ShapeDtypeStruct((E, 256), jnp.float32),
    )(edge_attr, hn[src], hn[dst], h2[src], h2[dst], w2pack)

    agg2 = _seg_sum(out2, dst, N)
    den_n2 = agg2[:, 3 * H]
    den_e2 = agg2[:, 3 * H + 1]
    nagg2 = agg2[:, :H] / (den_n2 + EPS)[:, None]
    eagg2 = (agg2[:, H:2 * H] @ W_np + agg2[:, 2 * H:3 * H] @ W_g) / (
        den_e2 + EPS)[:, None]
    x2 = _elu(jnp.concatenate([nagg2, eagg2], axis=1))

    sums = _seg_sum(x2, batch, NUM_GRAPHS)
    cnts = _seg_sum(jnp.ones((N,), jnp.float32), batch, NUM_GRAPHS)
    gpool = sums / jnp.maximum(cnts, 1.0)[:, None]
    return gpool @ Wr + br
